# baseline (speedup 1.0000x reference)
import jax
import jax.numpy as jnp
from jax.experimental import pallas as pl
from jax.experimental.pallas import tpu as pltpu

_BM = 400


def _gcn_kernel(x_ref, adj_ref, gc1_w_ref, gc1_b_ref, gc2_w_ref, gc2_b_ref,
                lin1_w_ref, lin1_b_ref, lin2_w_ref, lin2_b_ref,
                out_ref, sabu_s, sc_s, misc_s):
    j = pl.program_id(0)
    i = pl.program_id(1)
    bm = adj_ref.shape[0]
    nhid = gc1_w_ref.shape[1]
    rows = pl.ds(i * bm, bm)

    @pl.when(j == 0)
    def _proj():
        xx = x_ref[...]
        sA = jnp.dot(xx, gc1_w_ref[...], preferred_element_type=jnp.float32)
        l1 = jnp.dot(xx, lin1_w_ref[...], preferred_element_type=jnp.float32)
        l1 = l1 + lin1_b_ref[...]
        sB = jnp.dot(l1, gc2_w_ref[...], preferred_element_type=jnp.float32)
        ul = (jnp.dot(l1, lin2_w_ref[...], preferred_element_type=jnp.float32)
              + lin2_b_ref[...])
        sabu_s[rows, :] = jnp.concatenate([sA, sB, ul], axis=1)

    @pl.when(j == 1)
    def _pass1():
        h = jnp.dot(adj_ref[...], sabu_s[...],
                    preferred_element_type=jnp.float32)
        r1 = jnp.maximum(h[:, :nhid] + gc1_b_ref[...], 0.0)
        sc_s[rows, :] = jnp.dot(r1, gc2_w_ref[...],
                                preferred_element_type=jnp.float32)
        misc_s[rows, 0:8] = h[:, nhid:nhid + 8] + gc2_b_ref[...]
        misc_s[rows, 8:16] = (jnp.dot(r1, lin2_w_ref[...],
                                      preferred_element_type=jnp.float32)
                              + sabu_s[rows, nhid + 8:nhid + 16])

    @pl.when(j == 2)
    def _pass2():
        hc = jnp.dot(adj_ref[...], sc_s[...],
                     preferred_element_type=jnp.float32)
        out_ref[...] = (jnp.maximum(hc + misc_s[rows, 0:8], 0.0)
                        + misc_s[rows, 8:16])


@jax.jit
def kernel(x, adj, gc1_w, gc1_b, gc2_w, gc2_b,
           lin1_w, lin1_b, lin2_w, lin2_b):
    n, nfeat = x.shape
    nhid = gc1_w.shape[1]
    ncls = gc2_w.shape[1]

    full = lambda r, c: pl.BlockSpec((r, c), lambda j, i: (0, 0))

    out = pl.pallas_call(
        _gcn_kernel,
        grid=(3, n // _BM),
        in_specs=[
            pl.BlockSpec((_BM, nfeat),
                         lambda j, i: (jnp.where(j == 0, i, 0), 0)),
            pl.BlockSpec((_BM, n),
                         lambda j, i: (jnp.where(j == 0, 0, i), 0)),
            full(nfeat, nhid),
            full(1, nhid),
            full(nhid, ncls),
            full(1, ncls),
            full(nfeat, nhid),
            full(1, nhid),
            full(nhid, ncls),
            full(1, ncls),
        ],
        out_specs=pl.BlockSpec((_BM, ncls), lambda j, i: (i, 0)),
        out_shape=jax.ShapeDtypeStruct((n, ncls), jnp.float32),
        scratch_shapes=[
            pltpu.VMEM((n, nhid + 2 * ncls), jnp.float32),
            pltpu.VMEM((n, ncls), jnp.float32),
            pltpu.VMEM((n, 2 * ncls), jnp.float32),
        ],
        compiler_params=pltpu.CompilerParams(
            dimension_semantics=("arbitrary", "arbitrary"),
        ),
    )(x, adj, gc1_w, gc1_b.reshape(1, nhid), gc2_w, gc2_b.reshape(1, ncls),
      lin1_w, lin1_b.reshape(1, nhid), lin2_w, lin2_b.reshape(1, ncls))
    return out
